# baseline (speedup 1.0000x reference)
import functools

import jax
import jax.numpy as jnp
from jax import lax
from jax.experimental import pallas as pl
from jax.experimental.pallas import tpu as pltpu
from jax.experimental.pallas import tpu_sc as plsc

_NUM_CHARS = 100000
_HIDDEN = 16
_BATCH = 1024
_TM = 2000


@functools.cache
def _make_sc_gather():
    info = plsc.get_sparse_core_info()
    nc, ns = info.num_cores, info.num_subcores
    nw = nc * ns
    b_per_w = _BATCH // nw
    mesh = plsc.VectorSubcoreMesh(core_axis_name="c", subcore_axis_name="s")

    @functools.partial(
        pl.kernel,
        mesh=mesh,
        out_type=jax.ShapeDtypeStruct((_BATCH, _HIDDEN), jnp.float32),
        scratch_types=[
            pltpu.VMEM((b_per_w,), jnp.int32),
            pltpu.VMEM((b_per_w, _HIDDEN), jnp.float32),
            pltpu.SemaphoreType.DMA,
        ],
        compiler_params=pltpu.CompilerParams(use_tc_tiling_on_sc=False),
    )
    def gather_k(idx_hbm, table_hbm, out_hbm, idx_v, rows_v, sem):
        wid = lax.axis_index("s") * nc + lax.axis_index("c")
        base = wid * b_per_w
        pltpu.sync_copy(idx_hbm.at[pl.ds(base, b_per_w)], idx_v)
        pltpu.async_copy(table_hbm.at[idx_v], rows_v, sem).wait()
        pltpu.sync_copy(rows_v, out_hbm.at[pl.ds(base, b_per_w)])

    return gather_k


def _matmul_body(x_ref, proj_ref, out_ref):
    out_ref[...] = lax.dot_general(
        proj_ref[...],
        x_ref[...],
        dimension_numbers=(((1,), (1,)), ((), ())),
        preferred_element_type=jnp.float32,
    )


def _tc_matmul(x, proj):
    return pl.pallas_call(
        _matmul_body,
        grid=(_NUM_CHARS // _TM,),
        in_specs=[
            pl.BlockSpec((_BATCH, _HIDDEN), lambda i: (0, 0)),
            pl.BlockSpec((_TM, _HIDDEN), lambda i: (i, 0)),
        ],
        out_specs=pl.BlockSpec((_TM, _BATCH), lambda i: (i, 0)),
        out_shape=jax.ShapeDtypeStruct((_NUM_CHARS, _BATCH), jnp.float32),
    )(x, proj)


def kernel(indices, embedding, projection_matrix):
    x = _make_sc_gather()(indices.astype(jnp.int32), embedding)
    return _tc_matmul(x, projection_matrix)


# speedup vs baseline: 1.0221x; 1.0221x over previous
import functools

import jax
import jax.numpy as jnp
from jax import lax
from jax.experimental import pallas as pl
from jax.experimental.pallas import tpu as pltpu
from jax.experimental.pallas import tpu_sc as plsc

_NUM_CHARS = 100000
_HIDDEN = 16
_BATCH = 1024
_TM = 5000


@functools.cache
def _make_sc_gather():
    info = plsc.get_sparse_core_info()
    nc, ns = info.num_cores, info.num_subcores
    nw = nc * ns
    b_per_w = _BATCH // nw
    mesh = plsc.VectorSubcoreMesh(core_axis_name="c", subcore_axis_name="s")

    @functools.partial(
        pl.kernel,
        mesh=mesh,
        out_type=jax.ShapeDtypeStruct((_BATCH, _HIDDEN), jnp.float32),
        scratch_types=[
            pltpu.VMEM((b_per_w,), jnp.int32),
            pltpu.VMEM((b_per_w, _HIDDEN), jnp.float32),
            pltpu.SemaphoreType.DMA,
        ],
        compiler_params=pltpu.CompilerParams(use_tc_tiling_on_sc=False),
    )
    def gather_k(idx_hbm, table_hbm, out_hbm, idx_v, rows_v, sem):
        wid = lax.axis_index("s") * nc + lax.axis_index("c")
        base = wid * b_per_w
        pltpu.sync_copy(idx_hbm.at[pl.ds(base, b_per_w)], idx_v)
        pltpu.async_copy(table_hbm.at[idx_v], rows_v, sem).wait()
        pltpu.sync_copy(rows_v, out_hbm.at[pl.ds(base, b_per_w)])

    return gather_k


def _matmul_body(x_ref, proj_ref, out_ref):
    out_ref[...] = lax.dot_general(
        proj_ref[...],
        x_ref[...],
        dimension_numbers=(((1,), (1,)), ((), ())),
        preferred_element_type=jnp.float32,
    )


def _tc_matmul(x, proj):
    return pl.pallas_call(
        _matmul_body,
        grid=(_NUM_CHARS // _TM,),
        in_specs=[
            pl.BlockSpec((_BATCH, _HIDDEN), lambda i: (0, 0)),
            pl.BlockSpec((_TM, _HIDDEN), lambda i: (i, 0)),
        ],
        out_specs=pl.BlockSpec((_TM, _BATCH), lambda i: (i, 0)),
        out_shape=jax.ShapeDtypeStruct((_NUM_CHARS, _BATCH), jnp.float32),
    )(x, proj)


def kernel(indices, embedding, projection_matrix):
    x = _make_sc_gather()(indices.astype(jnp.int32), embedding)
    return _tc_matmul(x, projection_matrix)
